# baseline (speedup 1.0000x reference)
import functools

import jax
import jax.numpy as jnp
from jax.experimental import pallas as pl
from jax.experimental.pallas import tpu as pltpu

N = 16384
D_IN = 64
D_OUT = 64
BM = 256


def _support_kernel(x_ref, w_ref, out_ref):
    out_ref[...] = jnp.dot(x_ref[...], w_ref[...],
                           preferred_element_type=jnp.float32)


def _spmm_kernel(adj_ref, support_ref, bias_ref, out_ref):
    out_ref[...] = jnp.dot(adj_ref[...], support_ref[...],
                           preferred_element_type=jnp.float32) + bias_ref[...]


@jax.jit
def kernel(x, adj, weight, bias):
    support = pl.pallas_call(
        _support_kernel,
        out_shape=jax.ShapeDtypeStruct((N, D_OUT), jnp.float32),
    )(x, weight)

    bias2d = bias.reshape(1, D_OUT)
    out = pl.pallas_call(
        _spmm_kernel,
        grid=(N // BM,),
        in_specs=[
            pl.BlockSpec((BM, N), lambda i: (i, 0)),
            pl.BlockSpec((N, D_OUT), lambda i: (0, 0)),
            pl.BlockSpec((1, D_OUT), lambda i: (0, 0)),
        ],
        out_specs=pl.BlockSpec((BM, D_OUT), lambda i: (i, 0)),
        out_shape=jax.ShapeDtypeStruct((N, D_OUT), jnp.float32),
        compiler_params=pltpu.CompilerParams(
            dimension_semantics=("parallel",),
        ),
    )(adj, support, bias2d)
    return out


# speedup vs baseline: 1.0034x; 1.0034x over previous
import functools

import jax
import jax.numpy as jnp
from jax.experimental import pallas as pl
from jax.experimental.pallas import tpu as pltpu

N = 16384
D_IN = 64
D_OUT = 64
BM = 256


def _support_kernel(x_ref, w_ref, out_ref):
    out_ref[...] = jnp.dot(x_ref[...], w_ref[...],
                           preferred_element_type=jnp.float32
                           ).astype(jnp.bfloat16)


def _spmm_kernel(adj_ref, support_ref, bias_ref, out_ref):
    a = adj_ref[...].astype(jnp.bfloat16)
    out_ref[...] = jnp.dot(a, support_ref[...],
                           preferred_element_type=jnp.float32) + bias_ref[...]


@jax.jit
def kernel(x, adj, weight, bias):
    support = pl.pallas_call(
        _support_kernel,
        out_shape=jax.ShapeDtypeStruct((N, D_OUT), jnp.bfloat16),
    )(x, weight)

    bias2d = bias.reshape(1, D_OUT)
    out = pl.pallas_call(
        _spmm_kernel,
        grid=(N // BM,),
        in_specs=[
            pl.BlockSpec((BM, N), lambda i: (i, 0)),
            pl.BlockSpec((N, D_OUT), lambda i: (0, 0)),
            pl.BlockSpec((1, D_OUT), lambda i: (0, 0)),
        ],
        out_specs=pl.BlockSpec((BM, D_OUT), lambda i: (i, 0)),
        out_shape=jax.ShapeDtypeStruct((N, D_OUT), jnp.float32),
        compiler_params=pltpu.CompilerParams(
            dimension_semantics=("parallel",),
        ),
    )(adj, support, bias2d)
    return out
